# baseline (speedup 1.0000x reference)
import functools

import jax
import jax.numpy as jnp
from jax import lax
from jax.experimental import pallas as pl
from jax.experimental.pallas import tpu as pltpu
from jax.experimental.pallas import tpu_sc as plsc

NC = 2
NS = 16
L = 16
NW = NC * NS

NB = 2
W0 = 128


def kernel(x, table, scale):
    B, S = x.shape
    V, D = table.shape

    xi = x.astype(jnp.int32)
    scale_vec = jnp.broadcast_to(scale.astype(jnp.float32), (L,))
    t128 = jnp.concatenate([table, table], axis=1)

    b_per_w = B // NW
    n_chunks = b_per_w // NB
    s_lo = S - W0

    mesh = plsc.VectorSubcoreMesh(core_axis_name="c", subcore_axis_name="s")

    @functools.partial(
        pl.kernel,
        out_type=jax.ShapeDtypeStruct((B, S, D), jnp.float32),
        mesh=mesh,
        scratch_types=[
            pltpu.VMEM((1, S), jnp.int32),
            pltpu.VMEM((1, S), jnp.int32),
            pltpu.VMEM((NB, S, 2 * D), jnp.float32),
            pltpu.VMEM((NB, S, D), jnp.float32),
            pltpu.VMEM((L,), jnp.float32),
            pltpu.SemaphoreType.DMA,
        ],
        compiler_params=pltpu.CompilerParams(use_tc_tiling_on_sc=True),
    )
    def emb_kernel(idx_hbm, table_hbm, scale_hbm, out_hbm,
                   idx0, idx1, rows_v, out_v, scale_v, sem):
        wid = lax.axis_index("s") * NC + lax.axis_index("c")
        pltpu.sync_copy(scale_hbm, scale_v)
        sv = scale_v[...]

        @pl.loop(0, n_chunks)
        def _(ci):
            b0 = wid * b_per_w + ci * NB
            pltpu.sync_copy(idx_hbm.at[pl.ds(b0, 1)], idx0)
            pltpu.sync_copy(idx_hbm.at[pl.ds(b0 + 1, 1)], idx1)
            copies = []
            for i, idx_v in enumerate((idx0, idx1)):
                copies.append(pltpu.async_copy(
                    table_hbm.at[idx_v.at[0, pl.ds(0, W0)]],
                    rows_v.at[i, pl.ds(0, W0)], sem))
                copies.append(pltpu.async_copy(
                    table_hbm.at[idx_v.at[0, pl.ds(W0, s_lo)]],
                    rows_v.at[i, pl.ds(W0, s_lo)], sem))
            for c in copies:
                c.wait()

            @pl.loop(0, S)
            def _(r):
                for i in range(NB):
                    for jj in range(D // L):
                        sl = pl.ds(jj * L, L)
                        out_v[i, r, sl] = rows_v[i, r, sl] * sv

            pltpu.sync_copy(out_v, out_hbm.at[pl.ds(b0, NB)])

    return emb_kernel(xi, t128, scale_vec)
